# baseline (speedup 1.0000x reference)
import functools

import jax
import jax.numpy as jnp
from jax import lax
from jax.experimental import pallas as pl
from jax.experimental.pallas import tpu as pltpu
from jax.experimental.pallas import tpu_sc as plsc


def _pack_pairs(z_even, z_odd):
    hi = lax.bitcast_convert_type(
        z_even.astype(jnp.bfloat16), jnp.uint16).astype(jnp.uint32)
    lo = lax.bitcast_convert_type(
        z_odd.astype(jnp.bfloat16), jnp.uint16).astype(jnp.uint32)
    return lax.bitcast_convert_type((hi << 16) | lo, jnp.float32)


def _node_proj_body(w_ref, b_ref, x_ref, pt_ref):
    z = lax.dot_general(
        w_ref[...], x_ref[...], (((1,), (1,)), ((), ())),
        preferred_element_type=jnp.float32) + b_ref[...]
    H2 = z.shape[0] // 4
    ps = _pack_pairs(z[:H2], z[H2:2 * H2])
    pd = _pack_pairs(z[2 * H2:3 * H2], z[3 * H2:])
    pt_ref[...] = jnp.concatenate([ps, pd], axis=0)


def _edge_proj_body(w_ref, e_ref, qt_ref):
    z = lax.dot_general(
        w_ref[...], e_ref[...], (((1,), (1,)), ((), ())),
        preferred_element_type=jnp.float32)
    H2 = z.shape[0] // 2
    qt_ref[...] = _pack_pairs(z[:H2], z[H2:])


def _make_sc_combine(HP, NP, EC, NC, NS):
    NW = NC * NS
    G = EC // 16

    mesh = plsc.VectorSubcoreMesh(core_axis_name="c", subcore_axis_name="s")
    MASK_HI = jnp.uint32(0xFFFF0000)

    @functools.partial(
        pl.kernel,
        out_type=jax.ShapeDtypeStruct((NW * EC,), jnp.float32),
        mesh=mesh,
        compiler_params=pltpu.CompilerParams(needs_layout_passes=False),
        scratch_types=[
            pltpu.VMEM((EC,), jnp.int32),
            pltpu.VMEM((EC,), jnp.float32),
            pltpu.VMEM((NP,), jnp.float32),
            pltpu.VMEM((NP,), jnp.float32),
            pltpu.VMEM((NP,), jnp.float32),
            pltpu.VMEM((NP,), jnp.float32),
            pltpu.VMEM((EC,), jnp.float32),
            pltpu.VMEM((EC,), jnp.float32),
            pltpu.VMEM((2 * HP, 16), jnp.float32),
            pltpu.VMEM((16,), jnp.float32),
            pltpu.SemaphoreType.DMA,
            pltpu.SemaphoreType.DMA,
        ],
    )
    def sc_combine(pt_hbm, qt_hbm, sd_hbm, w2b_hbm, b2w_hbm,
                   out_hbm, sd_v, acc_v, ps_a, ps_b, pd_a, pd_b,
                   q_a, q_b, w2b_v, b2w_v, sem_a, sem_b):
        wid = lax.axis_index("s") * NC + lax.axis_index("c")
        base = wid * EC

        pltpu.sync_copy(sd_hbm.at[pl.ds(base, EC)], sd_v)
        pltpu.sync_copy(w2b_hbm, w2b_v)
        pltpu.sync_copy(b2w_hbm, b2w_v)

        bufs = [(ps_a, pd_a, q_a, sem_a), (ps_b, pd_b, q_b, sem_b)]

        def start(j, slot):
            ps, pd, q, sem = bufs[slot]
            return (
                pltpu.async_copy(pt_hbm.at[pl.ds(j * NP, NP)], ps, sem),
                pltpu.async_copy(pt_hbm.at[pl.ds((HP + j) * NP, NP)], pd, sem),
                pltpu.async_copy(qt_hbm.at[pl.ds(j * (NW * EC) + base, EC)],
                                 q, sem),
            )

        pending = start(0, 0)
        for j in range(HP):
            nxt = start(j + 1, (j + 1) % 2) if j + 1 < HP else None
            for c in pending:
                c.wait()
            ps, pd, q, _ = bufs[j % 2]
            w2e = w2b_v[j, :]
            w2o = w2b_v[HP + j, :]
            first = (j == 0)

            @plsc.parallel_loop(0, G, step=1, unroll=8)
            def _(g, _ps=ps, _pd=pd, _q=q, _w2e=w2e, _w2o=w2o, _first=first):
                off = pl.multiple_of(g * 16, 16)
                sd = sd_v[pl.ds(off, 16)]
                s = sd & 0xFFFF
                d = lax.shift_right_logical(sd, 16)
                wps = plsc.bitcast(plsc.load_gather(_ps, [s]), jnp.uint32)
                wpd = plsc.bitcast(plsc.load_gather(_pd, [d]), jnp.uint32)
                wq = plsc.bitcast(_q[pl.ds(off, 16)], jnp.uint32)
                pse = plsc.bitcast(wps & MASK_HI, jnp.float32)
                pso = plsc.bitcast(wps << 16, jnp.float32)
                pde = plsc.bitcast(wpd & MASK_HI, jnp.float32)
                pdo = plsc.bitcast(wpd << 16, jnp.float32)
                qe = plsc.bitcast(wq & MASK_HI, jnp.float32)
                qo = plsc.bitcast(wq << 16, jnp.float32)
                ze = jnp.maximum(pse + pde + qe, 0.0)
                zo = jnp.maximum(pso + pdo + qo, 0.0)
                upd = _w2e * ze + _w2o * zo
                if _first:
                    acc_v[pl.ds(off, 16)] = b2w_v[...] + upd
                else:
                    acc_v[pl.ds(off, 16)] = acc_v[pl.ds(off, 16)] + upd

            pending = nxt

        pltpu.sync_copy(acc_v, out_hbm.at[pl.ds(base, EC)])

    return sc_combine


def kernel(x, edge_index, e, W1, b1, W2, b2):
    N, D = x.shape
    E = e.shape[0]
    H = W1.shape[0]
    HP = H // 2

    info = plsc.get_sparse_core_info()
    NC, NS = info.num_cores, info.num_subcores
    NW = NC * NS

    BN = 1024
    BE = 3200
    NP = -(-N // BN) * BN
    unit = BE * NW * 16 // _gcd(BE, NW * 16)
    EP = -(-E // unit) * unit
    EC = EP // NW

    xp = jnp.pad(x, ((0, NP - N), (0, 0))) if NP != N else x
    ep = jnp.pad(e, ((0, EP - E), (0, 0))) if EP != E else e
    src = edge_index[0]
    dst = edge_index[1]
    if EP != E:
        src = jnp.pad(src, (0, EP - E))
        dst = jnp.pad(dst, (0, EP - E))
    sd = src | (dst << 16)

    perm = [2 * j for j in range(HP)] + [2 * j + 1 for j in range(HP)]
    W1s = W1[perm, :D]
    W1d = W1[perm, D:2 * D]
    W1e = W1[perm, 2 * D:]
    b1p = b1[jnp.asarray(perm)]
    Wsd = jnp.concatenate([W1s, W1d], axis=0)
    b1pad = jnp.concatenate([b1p, jnp.zeros_like(b1p)])[:, None]

    pt = pl.pallas_call(
        _node_proj_body,
        grid=(NP // BN,),
        in_specs=[
            pl.BlockSpec((2 * H, D), lambda i: (0, 0)),
            pl.BlockSpec((2 * H, 1), lambda i: (0, 0)),
            pl.BlockSpec((BN, D), lambda i: (i, 0)),
        ],
        out_specs=pl.BlockSpec((H, BN), lambda i: (0, i)),
        out_shape=jax.ShapeDtypeStruct((H, NP), jnp.float32),
    )(Wsd, b1pad, xp)

    qt = pl.pallas_call(
        _edge_proj_body,
        grid=(EP // BE,),
        in_specs=[
            pl.BlockSpec((H, D), lambda i: (0, 0)),
            pl.BlockSpec((BE, D), lambda i: (i, 0)),
        ],
        out_specs=pl.BlockSpec((HP, BE), lambda i: (0, i)),
        out_shape=jax.ShapeDtypeStruct((HP, EP), jnp.float32),
    )(W1e, ep)

    w2p = W2[0][jnp.asarray(perm)]
    w2b = jnp.broadcast_to(w2p[:, None], (H, 16)).astype(jnp.float32)
    b2w = jnp.broadcast_to(b2.astype(jnp.float32), (16,))

    sc = _make_sc_combine(HP, NP, EC, NC, NS)
    out = sc(pt.reshape(-1), qt.reshape(-1), sd, w2b, b2w)
    return out[:E, None]


def _gcd(a, b):
    while b:
        a, b = b, a % b
    return a


# speedup vs baseline: 1.8744x; 1.8744x over previous
import functools

import jax
import jax.numpy as jnp
from jax import lax
from jax.experimental import pallas as pl
from jax.experimental.pallas import tpu as pltpu
from jax.experimental.pallas import tpu_sc as plsc


def _pack_pairs(z_even, z_odd):
    hi = lax.bitcast_convert_type(
        z_even.astype(jnp.bfloat16), jnp.uint16).astype(jnp.uint32)
    lo = lax.bitcast_convert_type(
        z_odd.astype(jnp.bfloat16), jnp.uint16).astype(jnp.uint32)
    return lax.bitcast_convert_type((hi << 16) | lo, jnp.float32)


def _node_proj_body(w_ref, b_ref, x_ref, pt_ref):
    z = lax.dot_general(
        w_ref[...], x_ref[...], (((1,), (1,)), ((), ())),
        preferred_element_type=jnp.float32) + b_ref[...]
    H2 = z.shape[0] // 4
    ps = _pack_pairs(z[:H2], z[H2:2 * H2])
    pd = _pack_pairs(z[2 * H2:3 * H2], z[3 * H2:])
    pt_ref[...] = jnp.concatenate([ps, pd], axis=0)


def _edge_proj_body(w_ref, e_ref, qt_ref):
    z = lax.dot_general(
        w_ref[...], e_ref[...], (((1,), (1,)), ((), ())),
        preferred_element_type=jnp.float32)
    H2 = z.shape[0] // 2
    qt_ref[...] = _pack_pairs(z[:H2], z[H2:])


def _make_sc_combine(HP, NP, EC, NC, NS):
    NW = NC * NS
    G = EC // 16

    mesh = plsc.VectorSubcoreMesh(core_axis_name="c", subcore_axis_name="s")
    MASK_HI = jnp.uint32(0xFFFF0000)

    @functools.partial(
        pl.kernel,
        out_type=jax.ShapeDtypeStruct((NW * EC,), jnp.float32),
        mesh=mesh,
        compiler_params=pltpu.CompilerParams(needs_layout_passes=False),
        scratch_types=[
            pltpu.VMEM((EC,), jnp.int32),
            pltpu.VMEM((EC,), jnp.float32),
            pltpu.VMEM((NP,), jnp.float32),
            pltpu.VMEM((NP,), jnp.float32),
            pltpu.VMEM((NP,), jnp.float32),
            pltpu.VMEM((NP,), jnp.float32),
            pltpu.VMEM((EC,), jnp.float32),
            pltpu.VMEM((EC,), jnp.float32),
            pltpu.VMEM((2 * HP, 16), jnp.float32),
            pltpu.VMEM((16,), jnp.float32),
            pltpu.SemaphoreType.DMA,
            pltpu.SemaphoreType.DMA,
        ],
    )
    def sc_combine(pt_hbm, qt_hbm, sd_hbm, w2b_hbm, b2w_hbm,
                   out_hbm, sd_v, acc_v, ps_a, ps_b, pd_a, pd_b,
                   q_a, q_b, w2b_v, b2w_v, sem_a, sem_b):
        wid = lax.axis_index("s") * NC + lax.axis_index("c")
        base = wid * EC

        pltpu.sync_copy(sd_hbm.at[pl.ds(base, EC)], sd_v)
        pltpu.sync_copy(w2b_hbm, w2b_v)
        pltpu.sync_copy(b2w_hbm, b2w_v)

        bufs = [(ps_a, pd_a, q_a, sem_a), (ps_b, pd_b, q_b, sem_b)]

        def start(j, slot):
            ps, pd, q, sem = bufs[slot]
            return (
                pltpu.async_copy(pt_hbm.at[pl.ds(j * NP, NP)], ps, sem),
                pltpu.async_copy(pt_hbm.at[pl.ds((HP + j) * NP, NP)], pd, sem),
                pltpu.async_copy(qt_hbm.at[pl.ds(j * (NW * EC) + base, EC)],
                                 q, sem),
            )

        pending = start(0, 0)
        for j in range(HP):
            nxt = start(j + 1, (j + 1) % 2) if j + 1 < HP else None
            for c in pending:
                c.wait()
            ps, pd, q, _ = bufs[j % 2]
            w2e = w2b_v[j, :]
            w2o = w2b_v[HP + j, :]
            first = (j == 0)

            @plsc.parallel_loop(0, G, step=1, unroll=8)
            def _(g, _ps=ps, _pd=pd, _q=q, _w2e=w2e, _w2o=w2o, _first=first):
                off = pl.multiple_of(g * 16, 16)
                sd = sd_v[pl.ds(off, 16)]
                s = sd & 0xFFFF
                d = lax.shift_right_logical(sd, 16)
                wps = plsc.bitcast(plsc.load_gather(_ps, [s]), jnp.uint32)
                wpd = plsc.bitcast(plsc.load_gather(_pd, [d]), jnp.uint32)
                wq = plsc.bitcast(_q[pl.ds(off, 16)], jnp.uint32)
                pse = plsc.bitcast(wps & MASK_HI, jnp.float32)
                pso = plsc.bitcast(wps << 16, jnp.float32)
                pde = plsc.bitcast(wpd & MASK_HI, jnp.float32)
                pdo = plsc.bitcast(wpd << 16, jnp.float32)
                qe = plsc.bitcast(wq & MASK_HI, jnp.float32)
                qo = plsc.bitcast(wq << 16, jnp.float32)
                ze = jnp.maximum(pse + pde + qe, 0.0)
                zo = jnp.maximum(pso + pdo + qo, 0.0)
                upd = _w2e * ze + _w2o * zo
                if _first:
                    acc_v[pl.ds(off, 16)] = b2w_v[...] + upd
                else:
                    acc_v[pl.ds(off, 16)] = acc_v[pl.ds(off, 16)] + upd

            pending = nxt

        pltpu.sync_copy(acc_v, out_hbm.at[pl.ds(base, EC)])

    return sc_combine


def kernel(x, edge_index, e, W1, b1, W2, b2):
    N, D = x.shape
    E = e.shape[0]
    H = W1.shape[0]
    HP = H // 2

    info = plsc.get_sparse_core_info()
    NC, NS = info.num_cores, info.num_subcores
    NW = NC * NS

    BN = 1024
    BE = 3200
    NP = -(-N // BN) * BN
    unit = BE * NW * 16 // _gcd(BE, NW * 16)
    EP = -(-E // unit) * unit
    EC = EP // NW

    xp = jnp.pad(x, ((0, NP - N), (0, 0))) if NP != N else x
    ep = jnp.pad(e, ((0, EP - E), (0, 0))) if EP != E else e
    src = edge_index[0]
    dst = edge_index[1]
    if EP != E:
        src = jnp.pad(src, (0, EP - E))
        dst = jnp.pad(dst, (0, EP - E))
    sd = src | (dst << 16)

    def _perm_rows(a):
        r = a.reshape(HP, 2, *a.shape[1:])
        return jnp.concatenate([r[:, 0], r[:, 1]], axis=0)

    W1p = _perm_rows(W1)
    W1s = W1p[:, :D]
    W1d = W1p[:, D:2 * D]
    W1e = W1p[:, 2 * D:]
    b1p = _perm_rows(b1)
    Wsd = jnp.concatenate([W1s, W1d], axis=0)
    b1pad = jnp.concatenate([b1p, jnp.zeros_like(b1p)])[:, None]

    pt = pl.pallas_call(
        _node_proj_body,
        grid=(NP // BN,),
        in_specs=[
            pl.BlockSpec((2 * H, D), lambda i: (0, 0)),
            pl.BlockSpec((2 * H, 1), lambda i: (0, 0)),
            pl.BlockSpec((BN, D), lambda i: (i, 0)),
        ],
        out_specs=pl.BlockSpec((H, BN), lambda i: (0, i)),
        out_shape=jax.ShapeDtypeStruct((H, NP), jnp.float32),
    )(Wsd, b1pad, xp)

    qt = pl.pallas_call(
        _edge_proj_body,
        grid=(EP // BE,),
        in_specs=[
            pl.BlockSpec((H, D), lambda i: (0, 0)),
            pl.BlockSpec((BE, D), lambda i: (i, 0)),
        ],
        out_specs=pl.BlockSpec((HP, BE), lambda i: (0, i)),
        out_shape=jax.ShapeDtypeStruct((HP, EP), jnp.float32),
    )(W1e, ep)

    w2p = _perm_rows(W2[0])
    w2b = jnp.broadcast_to(w2p[:, None], (H, 16)).astype(jnp.float32)
    b2w = jnp.broadcast_to(b2.astype(jnp.float32), (16,))

    sc = _make_sc_combine(HP, NP, EC, NC, NS)
    out = sc(pt.reshape(-1), qt.reshape(-1), sd, w2b, b2w)
    return out[:E, None]


def _gcd(a, b):
    while b:
        a, b = b, a % b
    return a


# speedup vs baseline: 2.2167x; 1.1826x over previous
import functools

import jax
import jax.numpy as jnp
from jax import lax
from jax.experimental import pallas as pl
from jax.experimental.pallas import tpu as pltpu
from jax.experimental.pallas import tpu_sc as plsc


def _pack_pairs(z_even, z_odd):
    hi = lax.bitcast_convert_type(
        z_even.astype(jnp.bfloat16), jnp.uint16).astype(jnp.uint32)
    lo = lax.bitcast_convert_type(
        z_odd.astype(jnp.bfloat16), jnp.uint16).astype(jnp.uint32)
    return lax.bitcast_convert_type((hi << 16) | lo, jnp.float32)


def _node_proj_body(w_ref, b_ref, x_ref, pt_ref):
    z = lax.dot_general(
        w_ref[...], x_ref[...], (((1,), (1,)), ((), ())),
        preferred_element_type=jnp.float32) + b_ref[...]
    H2 = z.shape[0] // 4
    ps = _pack_pairs(z[:H2], z[H2:2 * H2])
    pd = _pack_pairs(z[2 * H2:3 * H2], z[3 * H2:])
    pt_ref[...] = jnp.concatenate([ps, pd], axis=0)


def _edge_proj_body(w_ref, e_ref, qt_ref):
    z = lax.dot_general(
        w_ref[...], e_ref[...], (((1,), (1,)), ((), ())),
        preferred_element_type=jnp.float32)
    H2 = z.shape[0] // 2
    qt_ref[...] = _pack_pairs(z[:H2], z[H2:])


def _make_sc_combine(HP, NP, EC, NC, NS):
    NW = NC * NS
    G = EC // 16

    mesh = plsc.VectorSubcoreMesh(core_axis_name="c", subcore_axis_name="s")
    MASK_HI = jnp.uint32(0xFFFF0000)

    @functools.partial(
        pl.kernel,
        out_type=jax.ShapeDtypeStruct((NW * EC,), jnp.float32),
        mesh=mesh,
        compiler_params=pltpu.CompilerParams(needs_layout_passes=False),
        scratch_types=[
            pltpu.VMEM((EC,), jnp.int32),
            pltpu.VMEM((EC,), jnp.int32),
            pltpu.VMEM((EC,), jnp.float32),
            pltpu.VMEM((NP,), jnp.float32),
            pltpu.VMEM((NP,), jnp.float32),
            pltpu.VMEM((NP,), jnp.float32),
            pltpu.VMEM((NP,), jnp.float32),
            pltpu.VMEM((EC,), jnp.float32),
            pltpu.VMEM((EC,), jnp.float32),
            pltpu.VMEM((2 * HP, 16), jnp.float32),
            pltpu.VMEM((16,), jnp.float32),
            pltpu.SemaphoreType.DMA,
            pltpu.SemaphoreType.DMA,
        ],
    )
    def sc_combine(pt_hbm, qt_hbm, src_hbm, dst_hbm, w2b_hbm, b2w_hbm,
                   out_hbm, src_v, dst_v, acc_v, ps_a, ps_b, pd_a, pd_b,
                   q_a, q_b, w2b_v, b2w_v, sem_a, sem_b):
        wid = lax.axis_index("s") * NC + lax.axis_index("c")
        base = wid * EC

        pltpu.sync_copy(src_hbm.at[pl.ds(base, EC)], src_v)
        pltpu.sync_copy(dst_hbm.at[pl.ds(base, EC)], dst_v)
        pltpu.sync_copy(w2b_hbm, w2b_v)
        pltpu.sync_copy(b2w_hbm, b2w_v)

        bufs = [(ps_a, pd_a, q_a, sem_a), (ps_b, pd_b, q_b, sem_b)]

        def start(j, slot):
            ps, pd, q, sem = bufs[slot]
            return (
                pltpu.async_copy(pt_hbm.at[pl.ds(j * NP, NP)], ps, sem),
                pltpu.async_copy(pt_hbm.at[pl.ds((HP + j) * NP, NP)], pd, sem),
                pltpu.async_copy(qt_hbm.at[pl.ds(j * (NW * EC) + base, EC)],
                                 q, sem),
            )

        pending = start(0, 0)
        for j in range(HP):
            nxt = start(j + 1, (j + 1) % 2) if j + 1 < HP else None
            for c in pending:
                c.wait()
            ps, pd, q, _ = bufs[j % 2]
            w2e = w2b_v[j, :]
            w2o = w2b_v[HP + j, :]
            first = (j == 0)

            @plsc.parallel_loop(0, G, step=1, unroll=8)
            def _(g, _ps=ps, _pd=pd, _q=q, _w2e=w2e, _w2o=w2o, _first=first):
                off = pl.multiple_of(g * 16, 16)
                s = src_v[pl.ds(off, 16)]
                d = dst_v[pl.ds(off, 16)]
                wps = plsc.bitcast(plsc.load_gather(_ps, [s]), jnp.uint32)
                wpd = plsc.bitcast(plsc.load_gather(_pd, [d]), jnp.uint32)
                wq = plsc.bitcast(_q[pl.ds(off, 16)], jnp.uint32)
                pse = plsc.bitcast(wps & MASK_HI, jnp.float32)
                pso = plsc.bitcast(wps << 16, jnp.float32)
                pde = plsc.bitcast(wpd & MASK_HI, jnp.float32)
                pdo = plsc.bitcast(wpd << 16, jnp.float32)
                qe = plsc.bitcast(wq & MASK_HI, jnp.float32)
                qo = plsc.bitcast(wq << 16, jnp.float32)
                ze = jnp.maximum(pse + pde + qe, 0.0)
                zo = jnp.maximum(pso + pdo + qo, 0.0)
                upd = _w2e * ze + _w2o * zo
                if _first:
                    acc_v[pl.ds(off, 16)] = b2w_v[...] + upd
                else:
                    acc_v[pl.ds(off, 16)] = acc_v[pl.ds(off, 16)] + upd

            pending = nxt

        pltpu.sync_copy(acc_v, out_hbm.at[pl.ds(base, EC)])

    return sc_combine


def kernel(x, edge_index, e, W1, b1, W2, b2):
    N, D = x.shape
    E = e.shape[0]
    H = W1.shape[0]
    HP = H // 2

    info = plsc.get_sparse_core_info()
    NC, NS = info.num_cores, info.num_subcores
    NW = NC * NS

    BN = 1024
    BE = 6400
    NP = -(-N // BN) * BN
    unit = BE * NW * 16 // _gcd(BE, NW * 16)
    EP = -(-E // unit) * unit
    EC = EP // NW

    xp = jnp.pad(x, ((0, NP - N), (0, 0))) if NP != N else x
    ep = jnp.pad(e, ((0, EP - E), (0, 0))) if EP != E else e
    src = edge_index[0]
    dst = edge_index[1]
    if EP != E:
        src = jnp.pad(src, (0, EP - E))
        dst = jnp.pad(dst, (0, EP - E))

    def _perm_rows(a):
        r = a.reshape(HP, 2, *a.shape[1:])
        return jnp.concatenate([r[:, 0], r[:, 1]], axis=0)

    W1p = _perm_rows(W1)
    W1s = W1p[:, :D]
    W1d = W1p[:, D:2 * D]
    W1e = W1p[:, 2 * D:]
    b1p = _perm_rows(b1)
    Wsd = jnp.concatenate([W1s, W1d], axis=0)
    b1pad = jnp.concatenate([b1p, jnp.zeros_like(b1p)])[:, None]

    pt = pl.pallas_call(
        _node_proj_body,
        grid=(NP // BN,),
        in_specs=[
            pl.BlockSpec((2 * H, D), lambda i: (0, 0)),
            pl.BlockSpec((2 * H, 1), lambda i: (0, 0)),
            pl.BlockSpec((BN, D), lambda i: (i, 0)),
        ],
        out_specs=pl.BlockSpec((H, BN), lambda i: (0, i)),
        out_shape=jax.ShapeDtypeStruct((H, NP), jnp.float32),
    )(Wsd, b1pad, xp)

    qt = pl.pallas_call(
        _edge_proj_body,
        grid=(EP // BE,),
        in_specs=[
            pl.BlockSpec((H, D), lambda i: (0, 0)),
            pl.BlockSpec((BE, D), lambda i: (i, 0)),
        ],
        out_specs=pl.BlockSpec((HP, BE), lambda i: (0, i)),
        out_shape=jax.ShapeDtypeStruct((HP, EP), jnp.float32),
    )(W1e, ep)

    w2p = _perm_rows(W2[0])
    w2b = jnp.broadcast_to(w2p[:, None], (H, 16)).astype(jnp.float32)
    b2w = jnp.broadcast_to(b2.astype(jnp.float32), (16,))

    sc = _make_sc_combine(HP, NP, EC, NC, NS)
    out = sc(pt.reshape(-1), qt.reshape(-1), src, dst, w2b, b2w)
    return out[:E, None]


def _gcd(a, b):
    while b:
        a, b = b, a % b
    return a


# speedup vs baseline: 2.5638x; 1.1565x over previous
import functools

import jax
import jax.numpy as jnp
from jax import lax
from jax.experimental import pallas as pl
from jax.experimental.pallas import tpu as pltpu
from jax.experimental.pallas import tpu_sc as plsc


def _pack_pairs(z_even, z_odd):
    hi = lax.bitcast_convert_type(
        z_even.astype(jnp.bfloat16), jnp.uint16).astype(jnp.uint32)
    lo = lax.bitcast_convert_type(
        z_odd.astype(jnp.bfloat16), jnp.uint16).astype(jnp.uint32)
    return lax.bitcast_convert_type((hi << 16) | lo, jnp.float32)


def _node_proj_body(w_ref, b_ref, x_ref, pt_ref):
    z = lax.dot_general(
        w_ref[...], x_ref[...], (((1,), (1,)), ((), ())),
        preferred_element_type=jnp.float32) + b_ref[...]
    H2 = z.shape[0] // 4
    ps = _pack_pairs(z[:H2], z[H2:2 * H2])
    pd = _pack_pairs(z[2 * H2:3 * H2], z[3 * H2:])
    pt_ref[...] = jnp.concatenate([ps, pd], axis=0)


def _edge_proj_body(w_ref, e_ref, qt_ref):
    z = lax.dot_general(
        w_ref[...], e_ref[...], (((1,), (1,)), ((), ())),
        preferred_element_type=jnp.float32)
    H2 = z.shape[0] // 2
    qt_ref[...] = _pack_pairs(z[:H2], z[H2:])


def _make_sc_combine(HP, NP, EC, NC, NS):
    NW = NC * NS
    G = EC // 16

    mesh = plsc.VectorSubcoreMesh(core_axis_name="c", subcore_axis_name="s")
    MASK_HI = jnp.uint32(0xFFFF0000)

    @functools.partial(
        pl.kernel,
        out_type=jax.ShapeDtypeStruct((NW * EC,), jnp.float32),
        mesh=mesh,
        compiler_params=pltpu.CompilerParams(needs_layout_passes=False),
        scratch_types=[
            pltpu.VMEM((EC,), jnp.int32),
            pltpu.VMEM((EC,), jnp.int32),
            pltpu.VMEM((EC,), jnp.int32),
            pltpu.VMEM((EC,), jnp.float32),
            pltpu.VMEM((NP,), jnp.float32),
            pltpu.VMEM((NP,), jnp.float32),
            pltpu.VMEM((NP,), jnp.float32),
            pltpu.VMEM((NP,), jnp.float32),
            pltpu.VMEM((EC,), jnp.float32),
            pltpu.VMEM((EC,), jnp.float32),
            pltpu.VMEM((2 * HP, 16), jnp.float32),
            pltpu.VMEM((16,), jnp.float32),
            pltpu.SemaphoreType.DMA,
            pltpu.SemaphoreType.DMA,
        ],
    )
    def sc_combine(ei_hbm, pt_hbm, qt_hbm, w2b_hbm, b2w_hbm,
                   out_hbm, src_v, dst_v, sd_v, acc_v, ps_a, ps_b, pd_a,
                   pd_b, q_a, q_b, w2b_v, b2w_v, sem_a, sem_b):
        wid = lax.axis_index("s") * NC + lax.axis_index("c")
        base = wid * EC

        pltpu.sync_copy(ei_hbm.at[pl.ds(base, EC)], src_v)
        pltpu.sync_copy(ei_hbm.at[pl.ds(NW * EC + base, EC)], dst_v)
        pltpu.sync_copy(w2b_hbm, w2b_v)
        pltpu.sync_copy(b2w_hbm, b2w_v)

        @plsc.parallel_loop(0, G, step=1, unroll=8)
        def _(g):
            off = pl.multiple_of(g * 16, 16)
            sd_v[pl.ds(off, 16)] = (src_v[pl.ds(off, 16)]
                                    | (dst_v[pl.ds(off, 16)] << 16))

        bufs = [(ps_a, pd_a, q_a, sem_a), (ps_b, pd_b, q_b, sem_b)]

        def start(j, slot):
            ps, pd, q, sem = bufs[slot]
            return (
                pltpu.async_copy(pt_hbm.at[pl.ds(j * NP, NP)], ps, sem),
                pltpu.async_copy(pt_hbm.at[pl.ds((HP + j) * NP, NP)], pd, sem),
                pltpu.async_copy(qt_hbm.at[pl.ds(j * (NW * EC) + base, EC)],
                                 q, sem),
            )

        pending = start(0, 0)
        for j in range(HP):
            nxt = start(j + 1, (j + 1) % 2) if j + 1 < HP else None
            for c in pending:
                c.wait()
            ps, pd, q, _ = bufs[j % 2]
            w2e = w2b_v[j, :]
            w2o = w2b_v[HP + j, :]
            first = (j == 0)

            @plsc.parallel_loop(0, G, step=1, unroll=8)
            def _(g, _ps=ps, _pd=pd, _q=q, _w2e=w2e, _w2o=w2o, _first=first):
                off = pl.multiple_of(g * 16, 16)
                sd = sd_v[pl.ds(off, 16)]
                s = sd & 0xFFFF
                d = lax.shift_right_logical(sd, 16)
                wps = plsc.bitcast(plsc.load_gather(_ps, [s]), jnp.uint32)
                wpd = plsc.bitcast(plsc.load_gather(_pd, [d]), jnp.uint32)
                wq = plsc.bitcast(_q[pl.ds(off, 16)], jnp.uint32)
                pse = plsc.bitcast(wps & MASK_HI, jnp.float32)
                pso = plsc.bitcast(wps << 16, jnp.float32)
                pde = plsc.bitcast(wpd & MASK_HI, jnp.float32)
                pdo = plsc.bitcast(wpd << 16, jnp.float32)
                qe = plsc.bitcast(wq & MASK_HI, jnp.float32)
                qo = plsc.bitcast(wq << 16, jnp.float32)
                ze = jnp.maximum(pse + pde + qe, 0.0)
                zo = jnp.maximum(pso + pdo + qo, 0.0)
                upd = _w2e * ze + _w2o * zo
                if _first:
                    acc_v[pl.ds(off, 16)] = b2w_v[...] + upd
                else:
                    acc_v[pl.ds(off, 16)] = acc_v[pl.ds(off, 16)] + upd

            pending = nxt

        pltpu.sync_copy(acc_v, out_hbm.at[pl.ds(base, EC)])

    return sc_combine


def kernel(x, edge_index, e, W1, b1, W2, b2):
    N, D = x.shape
    E = e.shape[0]
    H = W1.shape[0]
    HP = H // 2

    info = plsc.get_sparse_core_info()
    NC, NS = info.num_cores, info.num_subcores
    NW = NC * NS

    BN = 1024
    BE = 12800
    NP = -(-N // BN) * BN
    unit = BE * NW * 16 // _gcd(BE, NW * 16)
    EP = -(-E // unit) * unit
    EC = EP // NW

    xp = jnp.pad(x, ((0, NP - N), (0, 0))) if NP != N else x
    ep = jnp.pad(e, ((0, EP - E), (0, 0))) if EP != E else e
    if EP != E:
        ei = jnp.concatenate([jnp.pad(edge_index[0], (0, EP - E)),
                              jnp.pad(edge_index[1], (0, EP - E))])
    else:
        ei = edge_index.reshape(-1)

    def _perm_rows(a):
        r = a.reshape(HP, 2, *a.shape[1:])
        return jnp.concatenate([r[:, 0], r[:, 1]], axis=0)

    W1p = _perm_rows(W1)
    W1s = W1p[:, :D]
    W1d = W1p[:, D:2 * D]
    W1e = W1p[:, 2 * D:]
    b1p = _perm_rows(b1)
    Wsd = jnp.concatenate([W1s, W1d], axis=0)
    b1pad = jnp.concatenate([b1p, jnp.zeros_like(b1p)])[:, None]

    pt = pl.pallas_call(
        _node_proj_body,
        grid=(NP // BN,),
        in_specs=[
            pl.BlockSpec((2 * H, D), lambda i: (0, 0)),
            pl.BlockSpec((2 * H, 1), lambda i: (0, 0)),
            pl.BlockSpec((BN, D), lambda i: (i, 0)),
        ],
        out_specs=pl.BlockSpec((H, BN), lambda i: (0, i)),
        out_shape=jax.ShapeDtypeStruct((H, NP), jnp.float32),
    )(Wsd, b1pad, xp)

    qt = pl.pallas_call(
        _edge_proj_body,
        grid=(EP // BE,),
        in_specs=[
            pl.BlockSpec((H, D), lambda i: (0, 0)),
            pl.BlockSpec((BE, D), lambda i: (i, 0)),
        ],
        out_specs=pl.BlockSpec((HP, BE), lambda i: (0, i)),
        out_shape=jax.ShapeDtypeStruct((HP, EP), jnp.float32),
    )(W1e, ep)

    w2p = _perm_rows(W2[0])
    w2b = jnp.broadcast_to(w2p[:, None], (H, 16)).astype(jnp.float32)
    b2w = jnp.broadcast_to(b2.astype(jnp.float32), (16,))

    sc = _make_sc_combine(HP, NP, EC, NC, NS)
    out = sc(ei, pt.reshape(-1), qt.reshape(-1), w2b, b2w)
    return out[:E, None]


def _gcd(a, b):
    while b:
        a, b = b, a % b
    return a


# speedup vs baseline: 2.7295x; 1.0646x over previous
import functools

import jax
import jax.numpy as jnp
from jax import lax
from jax.experimental import pallas as pl
from jax.experimental.pallas import tpu as pltpu
from jax.experimental.pallas import tpu_sc as plsc


def _pack_pairs(z_even, z_odd):
    hi = lax.bitcast_convert_type(
        z_even.astype(jnp.bfloat16), jnp.uint16).astype(jnp.uint32)
    lo = lax.bitcast_convert_type(
        z_odd.astype(jnp.bfloat16), jnp.uint16).astype(jnp.uint32)
    return lax.bitcast_convert_type((hi << 16) | lo, jnp.float32)


def _node_proj_body(w_ref, b_ref, x_ref, pt_ref):
    z = lax.dot_general(
        w_ref[...], x_ref[...], (((1,), (1,)), ((), ())),
        preferred_element_type=jnp.float32) + b_ref[...]
    H2 = z.shape[0] // 4
    ps = _pack_pairs(z[:H2], z[H2:2 * H2])
    pd = _pack_pairs(z[2 * H2:3 * H2], z[3 * H2:])
    pt_ref[...] = jnp.concatenate([ps, pd], axis=0)


def _edge_proj_body(w_ref, e_ref, qt_ref):
    z = lax.dot_general(
        w_ref[...], e_ref[...], (((1,), (1,)), ((), ())),
        preferred_element_type=jnp.float32)
    H2 = z.shape[0] // 2
    qt_ref[...] = _pack_pairs(z[:H2], z[H2:])


def _make_sc_combine(HP, NP, EC, NC, NS):
    NW = NC * NS
    G = EC // 16

    mesh = plsc.VectorSubcoreMesh(core_axis_name="c", subcore_axis_name="s")
    MASK_HI = jnp.uint32(0xFFFF0000)

    @functools.partial(
        pl.kernel,
        out_type=jax.ShapeDtypeStruct((NW * EC,), jnp.float32),
        mesh=mesh,
        compiler_params=pltpu.CompilerParams(needs_layout_passes=False),
        scratch_types=[
            pltpu.VMEM((EC,), jnp.int32),
            pltpu.VMEM((EC,), jnp.int32),
            pltpu.VMEM((EC,), jnp.int32),
            pltpu.VMEM((EC,), jnp.float32),
            pltpu.VMEM((NP,), jnp.float32),
            pltpu.VMEM((NP,), jnp.float32),
            pltpu.VMEM((NP,), jnp.float32),
            pltpu.VMEM((NP,), jnp.float32),
            pltpu.VMEM((EC,), jnp.float32),
            pltpu.VMEM((EC,), jnp.float32),
            pltpu.VMEM((2 * HP, 16), jnp.float32),
            pltpu.VMEM((16,), jnp.float32),
            pltpu.SemaphoreType.DMA,
            pltpu.SemaphoreType.DMA,
        ],
    )
    def sc_combine(ei_hbm, pt_hbm, qt_hbm, w2b_hbm, b2w_hbm,
                   out_hbm, src_v, dst_v, sd_v, acc_v, ps_a, ps_b, pd_a,
                   pd_b, q_a, q_b, w2b_v, b2w_v, sem_a, sem_b):
        wid = lax.axis_index("s") * NC + lax.axis_index("c")
        base = wid * EC

        pltpu.sync_copy(ei_hbm.at[pl.ds(base, EC)], src_v)
        pltpu.sync_copy(ei_hbm.at[pl.ds(NW * EC + base, EC)], dst_v)
        pltpu.sync_copy(w2b_hbm, w2b_v)
        pltpu.sync_copy(b2w_hbm, b2w_v)

        b2vec = b2w_v[...]

        @plsc.parallel_loop(0, G, step=1, unroll=8)
        def _(g):
            off = pl.multiple_of(g * 16, 16)
            sd_v[pl.ds(off, 16)] = (src_v[pl.ds(off, 16)]
                                    | (dst_v[pl.ds(off, 16)] << 16))
            acc_v[pl.ds(off, 16)] = b2vec

        bufs = [(ps_a, pd_a, q_a, sem_a), (ps_b, pd_b, q_b, sem_b)]

        def start(j, slot):
            ps, pd, q, sem = bufs[slot]
            return (
                pltpu.async_copy(pt_hbm.at[pl.ds(j * NP, NP)], ps, sem),
                pltpu.async_copy(pt_hbm.at[pl.ds((HP + j) * NP, NP)], pd, sem),
                pltpu.async_copy(qt_hbm.at[pl.ds(j * (NW * EC) + base, EC)],
                                 q, sem),
            )

        pending = start(0, 0)
        for j in range(HP):
            nxt = start(j + 1, (j + 1) % 2) if j + 1 < HP else None
            for c in pending:
                c.wait()
            ps, pd, q, _ = bufs[j % 2]
            w2e = w2b_v[j, :]
            w2o = w2b_v[HP + j, :]

            @plsc.parallel_loop(0, G, step=1, unroll=8)
            def _(g, _ps=ps, _pd=pd, _q=q, _w2e=w2e, _w2o=w2o):
                off = pl.multiple_of(g * 16, 16)
                sd = sd_v[pl.ds(off, 16)]
                s = sd & 0xFFFF
                d = lax.shift_right_logical(sd, 16)
                wps = plsc.bitcast(plsc.load_gather(_ps, [s]), jnp.uint32)
                wpd = plsc.bitcast(plsc.load_gather(_pd, [d]), jnp.uint32)
                wq = plsc.bitcast(_q[pl.ds(off, 16)], jnp.uint32)
                pse = plsc.bitcast(wps & MASK_HI, jnp.float32)
                pso = plsc.bitcast(wps << 16, jnp.float32)
                pde = plsc.bitcast(wpd & MASK_HI, jnp.float32)
                pdo = plsc.bitcast(wpd << 16, jnp.float32)
                qe = plsc.bitcast(wq & MASK_HI, jnp.float32)
                qo = plsc.bitcast(wq << 16, jnp.float32)
                ze = jnp.maximum(pse + pde + qe, 0.0)
                zo = jnp.maximum(pso + pdo + qo, 0.0)
                upd = _w2e * ze + _w2o * zo
                plsc.addupdate(acc_v.at[pl.ds(off, 16)], upd)

            pending = nxt

        pltpu.sync_copy(acc_v, out_hbm.at[pl.ds(base, EC)])

    return sc_combine


def kernel(x, edge_index, e, W1, b1, W2, b2):
    N, D = x.shape
    E = e.shape[0]
    H = W1.shape[0]
    HP = H // 2

    info = plsc.get_sparse_core_info()
    NC, NS = info.num_cores, info.num_subcores
    NW = NC * NS

    BE = 12800
    NP = -(-N // 8) * 8
    unit = BE * NW * 16 // _gcd(BE, NW * 16)
    EP = -(-E // unit) * unit
    EC = EP // NW

    xp = jnp.pad(x, ((0, NP - N), (0, 0))) if NP != N else x
    ep = jnp.pad(e, ((0, EP - E), (0, 0))) if EP != E else e
    if EP != E:
        ei = jnp.concatenate([jnp.pad(edge_index[0], (0, EP - E)),
                              jnp.pad(edge_index[1], (0, EP - E))])
    else:
        ei = edge_index.reshape(-1)

    def _perm_rows(a):
        r = a.reshape(HP, 2, *a.shape[1:])
        return jnp.concatenate([r[:, 0], r[:, 1]], axis=0)

    W1p = _perm_rows(W1)
    W1s = W1p[:, :D]
    W1d = W1p[:, D:2 * D]
    W1e = W1p[:, 2 * D:]
    b1p = _perm_rows(b1)
    Wsd = jnp.concatenate([W1s, W1d], axis=0)
    b1pad = jnp.concatenate([b1p, jnp.zeros_like(b1p)])[:, None]

    pt = pl.pallas_call(
        _node_proj_body,
        out_shape=jax.ShapeDtypeStruct((H, NP), jnp.float32),
    )(Wsd, b1pad, xp)

    qt = pl.pallas_call(
        _edge_proj_body,
        grid=(EP // BE,),
        in_specs=[
            pl.BlockSpec((H, D), lambda i: (0, 0)),
            pl.BlockSpec((BE, D), lambda i: (i, 0)),
        ],
        out_specs=pl.BlockSpec((HP, BE), lambda i: (0, i)),
        out_shape=jax.ShapeDtypeStruct((HP, EP), jnp.float32),
    )(W1e, ep)

    w2p = _perm_rows(W2[0])
    w2b = jnp.broadcast_to(w2p[:, None], (H, 16)).astype(jnp.float32)
    b2w = jnp.broadcast_to(b2.astype(jnp.float32), (16,))

    sc = _make_sc_combine(HP, NP, EC, NC, NS)
    out = sc(ei, pt.reshape(-1), qt.reshape(-1), w2b, b2w)
    return out[:E, None]


def _gcd(a, b):
    while b:
        a, b = b, a % b
    return a
